# baseline (speedup 1.0000x reference)
import functools

import jax
import jax.numpy as jnp
from jax import lax
from jax.experimental import pallas as pl
from jax.experimental.pallas import tpu as pltpu, tpu_sc as plsc

N_CLS = 1000
N_CTX = 4
CTX_DIM = 768
SUF_LEN = 72
SEQ_LEN = 1 + N_CTX + SUF_LEN
ROW = CTX_DIM
OUT_C = SEQ_LEN * ROW
SUF_C = SUF_LEN * ROW
CTX_SZ = N_CTX * ROW


def _sc_concat(prefix_hbm, ctx_hbm, suffix_hbm, out_hbm, ctx_v, sem):
    info = plsc.get_sparse_core_info()
    nc = info.num_cores
    nw = nc * info.num_subcores
    wid = lax.axis_index("s") * nc + lax.axis_index("c")

    pltpu.sync_copy(ctx_hbm, ctx_v)

    per = N_CLS // nw
    rem = N_CLS - per * nw

    def start_class(c):
        base = pl.multiple_of(c * OUT_C, 8)
        return [
            pltpu.async_copy(
                prefix_hbm.at[pl.ds(pl.multiple_of(c * ROW, 8), ROW)],
                out_hbm.at[pl.ds(base, ROW)], sem),
            pltpu.async_copy(ctx_v, out_hbm.at[pl.ds(base + ROW, CTX_SZ)], sem),
            pltpu.async_copy(
                suffix_hbm.at[pl.ds(pl.multiple_of(c * SUF_C, 8), SUF_C)],
                out_hbm.at[pl.ds(base + ROW + CTX_SZ, SUF_C)], sem),
        ]

    copies = []
    for i in range(per):
        copies.extend(start_class(wid + nw * i))

    @pl.when(wid < rem)
    def _tail():
        for cp in start_class(per * nw + wid):
            cp.wait()

    for cp in copies:
        cp.wait()


@jax.jit
def kernel(token_prefix, ctx, token_suffix):
    mesh = plsc.VectorSubcoreMesh(core_axis_name="c", subcore_axis_name="s")
    fn = functools.partial(
        pl.kernel,
        mesh=mesh,
        out_type=jax.ShapeDtypeStruct((N_CLS * OUT_C,), jnp.float32),
        scratch_types=[pltpu.VMEM((CTX_SZ,), jnp.float32),
                       pltpu.SemaphoreType.DMA],
    )(_sc_concat)
    out = fn(token_prefix.reshape(-1), ctx.reshape(-1), token_suffix.reshape(-1))
    return out.reshape(N_CLS, SEQ_LEN, CTX_DIM)


# speedup vs baseline: 12.7549x; 12.7549x over previous
import functools

import jax
import jax.numpy as jnp
from jax import lax
from jax.experimental import pallas as pl
from jax.experimental.pallas import tpu as pltpu, tpu_sc as plsc

N_CLS = 1000
N_CTX = 4
CTX_DIM = 768
SUF_LEN = 72
SEQ_LEN = 1 + N_CTX + SUF_LEN
LANES = 16
CHUNKS = CTX_DIM // LANES


def _shift_and_head(buf, pbuf, cbuf):
    def body(k, carry):
        d = (SEQ_LEN - 1) - 2 * k
        for dd in (d, d - 1):
            for k16 in range(CHUNKS):
                buf[dd, pl.ds(k16 * LANES, LANES)] = \
                    buf[dd - 5, pl.ds(k16 * LANES, LANES)]
        return carry

    lax.fori_loop(0, SUF_LEN // 2, body, 0)

    for k16 in range(CHUNKS):
        buf[0, pl.ds(k16 * LANES, LANES)] = pbuf[0, pl.ds(k16 * LANES, LANES)]
    for r in range(N_CTX):
        for k16 in range(CHUNKS):
            buf[1 + r, pl.ds(k16 * LANES, LANES)] = \
                cbuf[pl.ds(r * CTX_DIM + k16 * LANES, LANES)]


def _sc_concat(prefix_hbm, ctx_hbm, suffix_hbm, out_hbm,
               buf0, buf1, pbuf0, pbuf1, cbuf, gsem0, gsem1, ssem0, ssem1):
    info = plsc.get_sparse_core_info()
    nc = info.num_cores
    nw = nc * info.num_subcores
    wid = lax.axis_index("s") * nc + lax.axis_index("c")

    pltpu.sync_copy(ctx_hbm, cbuf)

    per = N_CLS // nw
    rem = N_CLS - per * nw
    pairs = per // 2

    def g_start(c, buf, pbuf, gsem):
        return [
            pltpu.async_copy(suffix_hbm.at[c],
                             buf.at[pl.ds(0, SUF_LEN), :], gsem),
            pltpu.async_copy(prefix_hbm.at[c], pbuf, gsem),
        ]

    def s_start(c, buf, ssem):
        return pltpu.async_copy(buf, out_hbm.at[c], ssem)

    def g_wait(buf, pbuf, gsem):
        pltpu.make_async_copy(suffix_hbm.at[0],
                              buf.at[pl.ds(0, SUF_LEN), :], gsem).wait()
        pltpu.make_async_copy(prefix_hbm.at[0], pbuf, gsem).wait()

    def s_wait(buf, ssem):
        pltpu.make_async_copy(buf, out_hbm.at[0], ssem).wait()

    def process(c, buf, pbuf):
        _shift_and_head(buf, pbuf, cbuf)

    def body(j, carry):
        a = wid + nw * (2 * j)
        b = wid + nw * (2 * j + 1)

        @pl.when(j > 0)
        def _w0():
            s_wait(buf0, ssem0)
        g_start(a, buf0, pbuf0, gsem0)

        @pl.when(j > 0)
        def _w1():
            s_wait(buf1, ssem1)
        g_start(b, buf1, pbuf1, gsem1)

        g_wait(buf0, pbuf0, gsem0)
        process(a, buf0, pbuf0)
        s_start(a, buf0, ssem0)

        g_wait(buf1, pbuf1, gsem1)
        process(b, buf1, pbuf1)
        s_start(b, buf1, ssem1)
        return carry

    lax.fori_loop(0, pairs, body, 0)
    s_wait(buf0, ssem0)
    s_wait(buf1, ssem1)

    c_last = wid + nw * (per - 1)
    for cp in g_start(c_last, buf0, pbuf0, gsem0):
        cp.wait()
    process(c_last, buf0, pbuf0)
    s_start(c_last, buf0, ssem0).wait()

    @pl.when(wid < rem)
    def _tail():
        c = per * nw + wid
        for cp in g_start(c, buf1, pbuf1, gsem1):
            cp.wait()
        process(c, buf1, pbuf1)
        s_start(c, buf1, ssem1).wait()


@jax.jit
def kernel(token_prefix, ctx, token_suffix):
    mesh = plsc.VectorSubcoreMesh(core_axis_name="c", subcore_axis_name="s")
    fn = functools.partial(
        pl.kernel,
        mesh=mesh,
        out_type=jax.ShapeDtypeStruct((N_CLS, SEQ_LEN, CTX_DIM), jnp.float32),
        scratch_types=[pltpu.VMEM((SEQ_LEN, CTX_DIM), jnp.float32),
                       pltpu.VMEM((SEQ_LEN, CTX_DIM), jnp.float32),
                       pltpu.VMEM((1, CTX_DIM), jnp.float32),
                       pltpu.VMEM((1, CTX_DIM), jnp.float32),
                       pltpu.VMEM((N_CTX * CTX_DIM,), jnp.float32),
                       pltpu.SemaphoreType.DMA,
                       pltpu.SemaphoreType.DMA,
                       pltpu.SemaphoreType.DMA,
                       pltpu.SemaphoreType.DMA],
    )(_sc_concat)
    return fn(token_prefix, ctx.reshape(-1), token_suffix)
